# Initial kernel scaffold; baseline (speedup 1.0000x reference)
import jax
import jax.numpy as jnp
from jax.experimental import pallas as pl


def kernel(weight, qlen, klen):
    raise NotImplementedError("write your pallas kernel here")


# speedup vs baseline: 42.0822x; 42.0822x over previous
import functools
import math

import jax
import jax.numpy as jnp
from jax import lax
from jax.experimental import pallas as pl
from jax.experimental.pallas import tpu as pltpu
from jax.experimental.pallas import tpu_sc as plsc

_NUM_BUCKETS = 32
_MAX_DISTANCE = 128
_N_HEADS = 16
_QLEN = 2048
_KLEN = 2048
_NSHIFT = 16
_TEXT = 4224
_TWIDTH = 4096


def _table_body(delta_ref, w_ref, out_ref):
    m = lax.broadcasted_iota(jnp.int32, (1, _TEXT), 1)
    rel = m - (_QLEN - 1) + delta_ref[0, 0]
    n = -rel
    half = _NUM_BUCKETS // 2
    ret = jnp.where(n < 0, half, 0).astype(jnp.int32)
    n = jnp.abs(n)
    max_exact = half // 2
    is_small = n < max_exact
    nf = n.astype(jnp.float32)
    val_if_large = max_exact + (
        jnp.log(nf / max_exact)
        / math.log(_MAX_DISTANCE / max_exact)
        * (half - max_exact)
    ).astype(jnp.int32)
    val_if_large = jnp.minimum(val_if_large, half - 1)
    bucket = ret + jnp.where(is_small, n, val_if_large)

    onehot = jnp.equal(
        lax.broadcasted_iota(jnp.int32, (_NUM_BUCKETS, _TEXT), 0), bucket
    ).astype(jnp.float32)
    t_ext = lax.dot_general(
        w_ref[...], onehot, (((0,), (0,)), ((), ())),
        preferred_element_type=jnp.float32,
        precision=lax.Precision.HIGHEST,
    )
    for r in range(_NSHIFT):
        out_ref[:, r, :] = t_ext[:, r:r + _TWIDTH]


def _make_table(weight, delta):
    return pl.pallas_call(
        _table_body,
        out_shape=jax.ShapeDtypeStruct((_N_HEADS, _NSHIFT, _TWIDTH), jnp.float32),
        in_specs=[
            pl.BlockSpec(memory_space=pltpu.SMEM),
            pl.BlockSpec(memory_space=pltpu.VMEM),
        ],
        out_specs=pl.BlockSpec(memory_space=pltpu.VMEM),
    )(delta, weight)


def _writer_body(t16_hbm, out_hbm, t16_v, sem):
    wid = lax.axis_index("s") * 2 + lax.axis_index("c")
    head = wid // 2
    row0 = (wid % 2) * 1024
    pltpu.sync_copy(t16_hbm.at[head], t16_v)

    def fire(i, carry):
        irow = row0 + i
        s = (_KLEN - 1) - irow
        r = jnp.bitwise_and(s, _NSHIFT - 1)
        base = pl.multiple_of(jnp.bitwise_and(s, -_NSHIFT), _NSHIFT)
        pltpu.make_async_copy(
            t16_v.at[r, pl.ds(base, _KLEN)], out_hbm.at[head, irow], sem
        ).start()
        return carry

    lax.fori_loop(0, 1024, fire, 0)

    def drain(i, carry):
        pltpu.make_async_copy(
            t16_v.at[0, pl.ds(0, _KLEN)], out_hbm.at[head, row0], sem
        ).wait()
        return carry

    lax.fori_loop(0, 1024, drain, 0)


@functools.cache
def _writer():
    return pl.kernel(
        _writer_body,
        out_type=jax.ShapeDtypeStruct((_N_HEADS, _QLEN, _KLEN), jnp.float32),
        mesh=plsc.VectorSubcoreMesh(core_axis_name="c", subcore_axis_name="s"),
        scratch_types=[
            pltpu.VMEM((_NSHIFT, _TWIDTH), jnp.float32),
            pltpu.SemaphoreType.DMA,
        ],
        compiler_params=pltpu.CompilerParams(use_tc_tiling_on_sc=False),
    )


def kernel(weight, qlen, klen):
    delta = (jnp.asarray(klen, jnp.int32) - jnp.asarray(qlen, jnp.int32))
    t16 = _make_table(weight, delta.reshape(1, 1))
    out = _writer()(t16)
    return out.reshape(1, _N_HEADS, _QLEN, _KLEN)
